# Initial kernel scaffold; baseline (speedup 1.0000x reference)
import jax
import jax.numpy as jnp
from jax.experimental import pallas as pl


def kernel(x, Ws, bs, scaling_params):
    raise NotImplementedError("write your pallas kernel here")


# speedup vs baseline: 2.2033x; 2.2033x over previous
import functools

import jax
import jax.numpy as jnp
from jax import lax
from jax.experimental import pallas as pl
from jax.experimental.pallas import tpu as pltpu
from jax.experimental.pallas import tpu_sc as plsc

_K = 2
_LANES = 16
_BN = 512


def _routing_body(sp_hbm, bs_hbm, sp_v, bs_v):
    is_lead = (lax.axis_index("c") == 0) & (lax.axis_index("s") == 0)

    pltpu.sync_copy(sp_hbm, sp_v)
    s = sp_v[...]
    iota = lax.iota(jnp.int32, _LANES)
    neg_inf = jnp.float32(-jnp.inf)

    def _bfly(v, op):
        for k in (1, 2, 4, 8):
            v = op(v, v.at[iota ^ k].get(mode="promise_in_bounds"))
        return v

    m0 = _bfly(s, jnp.maximum)
    i0 = _bfly(jnp.where(s == m0, iota, jnp.int32(_LANES)), jnp.minimum)
    s1 = jnp.where(iota == i0, neg_inf, s)
    m1 = _bfly(s1, jnp.maximum)
    i1 = _bfly(jnp.where(s1 == m1, iota, jnp.int32(_LANES)), jnp.minimum)

    e1 = jnp.exp(m1 - m0)
    denom = jnp.float32(1.0) + e1
    p0 = jnp.float32(1.0) / denom
    p1 = e1 / denom

    zero = jnp.float32(0.0)
    bs_v[...] = jnp.where(iota == i0, p0, jnp.where(iota == i1, p1, zero))

    @pl.when(is_lead)
    def _():
        pltpu.sync_copy(bs_v, bs_hbm)


def _sc_routing(sp16):
    run = pl.kernel(
        _routing_body,
        mesh=plsc.VectorSubcoreMesh(core_axis_name="c", subcore_axis_name="s"),
        out_type=jax.ShapeDtypeStruct((_LANES,), jnp.float32),
        scratch_types=[
            pltpu.VMEM((_LANES,), jnp.float32),
            pltpu.VMEM((_LANES,), jnp.float32),
        ],
    )
    return run(sp16)


def _mm_body(sp_ref, x_ref, ws_ref, bs_ref, o_ref,
             w0_ref, w1_ref, wc_ref, bc_ref, sem0, sem1):
    i = pl.program_id(0)
    e = bs_ref.shape[0]

    m0 = sp_ref[0]
    i0 = jnp.int32(0)
    for j in range(1, e):
        better = sp_ref[j] > m0
        m0 = jnp.where(better, sp_ref[j], m0)
        i0 = jnp.where(better, jnp.int32(j), i0)
    m1 = jnp.float32(-jnp.inf)
    i1 = jnp.int32(0)
    for j in range(e):
        better = (jnp.int32(j) != i0) & (sp_ref[j] > m1)
        m1 = jnp.where(better, sp_ref[j], m1)
        i1 = jnp.where(better, jnp.int32(j), i1)

    ev = jnp.exp(jnp.full((8, 128), m1 - m0, dtype=jnp.float32))
    e1 = ev[0, 0]
    denom = jnp.float32(1.0) + e1
    p0 = jnp.float32(1.0) / denom
    p1 = e1 / denom

    @pl.when(i == 0)
    def _():
        c0 = pltpu.make_async_copy(ws_ref.at[i0], w0_ref, sem0)
        c1 = pltpu.make_async_copy(ws_ref.at[i1], w1_ref, sem1)
        c0.start()
        c1.start()
        c0.wait()
        c1.wait()
        wc_ref[...] = (p0 * w0_ref[...] + p1 * w1_ref[...]).astype(jnp.bfloat16)
        b = jnp.zeros((1, bs_ref.shape[1]), jnp.float32)
        for j in range(e):
            pj = jnp.where(i0 == j, p0, jnp.where(i1 == j, p1, jnp.float32(0.0)))
            b = b + pj * bs_ref[j][None, :]
        bc_ref[...] = b

    acc = lax.dot_general(
        x_ref[...].astype(jnp.bfloat16), wc_ref[...],
        dimension_numbers=(((1,), (1,)), ((), ())),
        preferred_element_type=jnp.float32,
    )
    o_ref[...] = acc + bc_ref[...]


def _tc_mixture(x, Ws, bs, sp):
    n, d = x.shape
    e = Ws.shape[0]
    grid = (n // _BN,)
    return pl.pallas_call(
        _mm_body,
        grid=grid,
        in_specs=[
            pl.BlockSpec(memory_space=pltpu.SMEM),
            pl.BlockSpec((_BN, d), lambda i: (i, 0)),
            pl.BlockSpec(memory_space=pl.ANY),
            pl.BlockSpec(memory_space=pltpu.VMEM),
        ],
        out_specs=pl.BlockSpec((_BN, d), lambda i: (i, 0)),
        out_shape=jax.ShapeDtypeStruct((n, d), jnp.float32),
        scratch_shapes=[
            pltpu.VMEM((d, d), jnp.float32),
            pltpu.VMEM((d, d), jnp.float32),
            pltpu.VMEM((d, d), jnp.bfloat16),
            pltpu.VMEM((1, d), jnp.float32),
            pltpu.SemaphoreType.DMA,
            pltpu.SemaphoreType.DMA,
        ],
    )(sp, x, Ws, bs)


def kernel(x, Ws, bs, scaling_params):
    e, d, _ = Ws.shape
    sp = scaling_params.astype(jnp.float32)
    sp16 = jnp.full((_LANES,), -jnp.inf, dtype=jnp.float32)
    sp16 = sp16.at[:e].set(sp)

    bs16 = _sc_routing(sp16)
    backbone_scores = bs16[:e].astype(x.dtype)

    out = _tc_mixture(x, Ws, bs, sp)
    return out, backbone_scores


# speedup vs baseline: 2.3922x; 1.0857x over previous
import functools

import jax
import jax.numpy as jnp
from jax import lax
from jax.experimental import pallas as pl
from jax.experimental.pallas import tpu as pltpu
from jax.experimental.pallas import tpu_sc as plsc

_K = 2
_LANES = 16
_BN = 2048


def _routing_body(sp_hbm, bs_hbm, sp_v, bs_v):
    is_lead = (lax.axis_index("c") == 0) & (lax.axis_index("s") == 0)

    pltpu.sync_copy(sp_hbm, sp_v)
    s = sp_v[...]
    iota = lax.iota(jnp.int32, _LANES)
    neg_inf = jnp.float32(-jnp.inf)

    def _bfly(v, op):
        for k in (1, 2, 4, 8):
            v = op(v, v.at[iota ^ k].get(mode="promise_in_bounds"))
        return v

    m0 = _bfly(s, jnp.maximum)
    i0 = _bfly(jnp.where(s == m0, iota, jnp.int32(_LANES)), jnp.minimum)
    s1 = jnp.where(iota == i0, neg_inf, s)
    m1 = _bfly(s1, jnp.maximum)
    i1 = _bfly(jnp.where(s1 == m1, iota, jnp.int32(_LANES)), jnp.minimum)

    e1 = jnp.exp(m1 - m0)
    denom = jnp.float32(1.0) + e1
    p0 = jnp.float32(1.0) / denom
    p1 = e1 / denom

    zero = jnp.float32(0.0)
    bs_v[...] = jnp.where(iota == i0, p0, jnp.where(iota == i1, p1, zero))

    @pl.when(is_lead)
    def _():
        pltpu.sync_copy(bs_v, bs_hbm)


def _sc_routing(sp16):
    run = pl.kernel(
        _routing_body,
        mesh=plsc.VectorSubcoreMesh(core_axis_name="c", subcore_axis_name="s"),
        out_type=jax.ShapeDtypeStruct((_LANES,), jnp.float32),
        scratch_types=[
            pltpu.VMEM((_LANES,), jnp.float32),
            pltpu.VMEM((_LANES,), jnp.float32),
        ],
    )
    return run(sp16)


def _mm_body(sp_ref, x_ref, ws_ref, bs_ref, o_ref,
             w0_ref, w1_ref, wc_ref, bc_ref, sem0, sem1):
    i = pl.program_id(0)
    e = bs_ref.shape[0]

    m0 = sp_ref[0]
    i0 = jnp.int32(0)
    for j in range(1, e):
        better = sp_ref[j] > m0
        m0 = jnp.where(better, sp_ref[j], m0)
        i0 = jnp.where(better, jnp.int32(j), i0)
    m1 = jnp.float32(-jnp.inf)
    i1 = jnp.int32(0)
    for j in range(e):
        better = (jnp.int32(j) != i0) & (sp_ref[j] > m1)
        m1 = jnp.where(better, sp_ref[j], m1)
        i1 = jnp.where(better, jnp.int32(j), i1)

    ev = jnp.exp(jnp.full((8, 128), m1 - m0, dtype=jnp.float32))
    e1 = ev[0, 0]
    denom = jnp.float32(1.0) + e1
    p0 = jnp.float32(1.0) / denom
    p1 = e1 / denom

    @pl.when(i == 0)
    def _():
        c0 = pltpu.make_async_copy(ws_ref.at[i0], w0_ref, sem0)
        c1 = pltpu.make_async_copy(ws_ref.at[i1], w1_ref, sem1)
        c0.start()
        c1.start()
        c0.wait()
        c1.wait()
        wc_ref[...] = (p0 * w0_ref[...] + p1 * w1_ref[...]).astype(jnp.bfloat16)
        b = jnp.zeros((1, bs_ref.shape[1]), jnp.float32)
        for j in range(e):
            pj = jnp.where(i0 == j, p0, jnp.where(i1 == j, p1, jnp.float32(0.0)))
            b = b + pj * bs_ref[j][None, :]
        bc_ref[...] = b

    acc = lax.dot_general(
        x_ref[...].astype(jnp.bfloat16), wc_ref[...],
        dimension_numbers=(((1,), (1,)), ((), ())),
        preferred_element_type=jnp.float32,
    )
    o_ref[...] = acc + bc_ref[...]


def _tc_mixture(x, Ws, bs, sp):
    n, d = x.shape
    e = Ws.shape[0]
    grid = (n // _BN,)
    return pl.pallas_call(
        _mm_body,
        grid=grid,
        in_specs=[
            pl.BlockSpec(memory_space=pltpu.SMEM),
            pl.BlockSpec((_BN, d), lambda i: (i, 0)),
            pl.BlockSpec(memory_space=pl.ANY),
            pl.BlockSpec(memory_space=pltpu.VMEM),
        ],
        out_specs=pl.BlockSpec((_BN, d), lambda i: (i, 0)),
        out_shape=jax.ShapeDtypeStruct((n, d), jnp.float32),
        scratch_shapes=[
            pltpu.VMEM((d, d), jnp.float32),
            pltpu.VMEM((d, d), jnp.float32),
            pltpu.VMEM((d, d), jnp.bfloat16),
            pltpu.VMEM((1, d), jnp.float32),
            pltpu.SemaphoreType.DMA,
            pltpu.SemaphoreType.DMA,
        ],
    )(sp, x, Ws, bs)


def kernel(x, Ws, bs, scaling_params):
    e, d, _ = Ws.shape
    sp = scaling_params.astype(jnp.float32)
    sp16 = jnp.full((_LANES,), -jnp.inf, dtype=jnp.float32)
    sp16 = sp16.at[:e].set(sp)

    bs16 = _sc_routing(sp16)
    backbone_scores = bs16[:e].astype(x.dtype)

    out = _tc_mixture(x, Ws, bs, sp)
    return out, backbone_scores


# speedup vs baseline: 3.6053x; 1.5071x over previous
import functools

import jax
import jax.numpy as jnp
from jax import lax
from jax.experimental import pallas as pl
from jax.experimental.pallas import tpu as pltpu
from jax.experimental.pallas import tpu_sc as plsc

_K = 2
_LANES = 16
_BN = 2048


def _routing_body(sp_hbm, bs_hbm, sp_v, bs_v):
    is_lead = (lax.axis_index("c") == 0) & (lax.axis_index("s") == 0)

    pltpu.sync_copy(sp_hbm, sp_v)
    s = sp_v[...]
    iota = lax.iota(jnp.int32, _LANES)
    neg_inf = jnp.float32(-jnp.inf)

    def _bfly(v, op):
        for k in (1, 2, 4, 8):
            v = op(v, v.at[iota ^ k].get(mode="promise_in_bounds"))
        return v

    m0 = _bfly(s, jnp.maximum)
    i0 = _bfly(jnp.where(s == m0, iota, jnp.int32(_LANES)), jnp.minimum)
    s1 = jnp.where(iota == i0, neg_inf, s)
    m1 = _bfly(s1, jnp.maximum)
    i1 = _bfly(jnp.where(s1 == m1, iota, jnp.int32(_LANES)), jnp.minimum)

    e1 = jnp.exp(m1 - m0)
    denom = jnp.float32(1.0) + e1
    p0 = jnp.float32(1.0) / denom
    p1 = e1 / denom

    zero = jnp.float32(0.0)
    bs_v[...] = jnp.where(iota == i0, p0, jnp.where(iota == i1, p1, zero))

    @pl.when(is_lead)
    def _():
        pltpu.sync_copy(bs_v, bs_hbm)


def _sc_routing(sp16):
    run = pl.kernel(
        _routing_body,
        mesh=plsc.VectorSubcoreMesh(core_axis_name="c", subcore_axis_name="s"),
        out_type=jax.ShapeDtypeStruct((_LANES,), jnp.float32),
        scratch_types=[
            pltpu.VMEM((_LANES,), jnp.float32),
            pltpu.VMEM((_LANES,), jnp.float32),
        ],
    )
    return run(sp16)


def _mm_body(sp_ref, x_ref, ws_ref, bs_ref, o_ref, bb_ref,
             w0_ref, w1_ref, wc_ref, bc_ref, sem0, sem1):
    i = pl.program_id(0)
    e = bs_ref.shape[0]

    m0 = sp_ref[0]
    i0 = jnp.int32(0)
    for j in range(1, e):
        better = sp_ref[j] > m0
        m0 = jnp.where(better, sp_ref[j], m0)
        i0 = jnp.where(better, jnp.int32(j), i0)
    m1 = jnp.float32(-jnp.inf)
    i1 = jnp.int32(0)
    for j in range(e):
        better = (jnp.int32(j) != i0) & (sp_ref[j] > m1)
        m1 = jnp.where(better, sp_ref[j], m1)
        i1 = jnp.where(better, jnp.int32(j), i1)

    ev = jnp.exp(jnp.full((8, 128), m1 - m0, dtype=jnp.float32))
    e1 = ev[0, 0]
    denom = jnp.float32(1.0) + e1
    p0 = jnp.float32(1.0) / denom
    p1 = e1 / denom

    @pl.when(i == 0)
    def _():
        c0 = pltpu.make_async_copy(ws_ref.at[i0], w0_ref, sem0)
        c1 = pltpu.make_async_copy(ws_ref.at[i1], w1_ref, sem1)
        c0.start()
        c1.start()
        c0.wait()
        c1.wait()
        wc_ref[...] = (p0 * w0_ref[...] + p1 * w1_ref[...]).astype(jnp.bfloat16)
        b = jnp.zeros((1, bs_ref.shape[1]), jnp.float32)
        for j in range(e):
            pj = jnp.where(i0 == j, p0, jnp.where(i1 == j, p1, jnp.float32(0.0)))
            b = b + pj * bs_ref[j][None, :]
        bc_ref[...] = b
        for j in range(e):
            bb_ref[j] = jnp.where(i0 == j, p0,
                                  jnp.where(i1 == j, p1, jnp.float32(0.0)))

    acc = lax.dot_general(
        x_ref[...].astype(jnp.bfloat16), wc_ref[...],
        dimension_numbers=(((1,), (1,)), ((), ())),
        preferred_element_type=jnp.float32,
    )
    o_ref[...] = acc + bc_ref[...]


def _tc_mixture(x, Ws, bs, sp):
    n, d = x.shape
    e = Ws.shape[0]
    grid = (n // _BN,)
    return pl.pallas_call(
        _mm_body,
        grid=grid,
        in_specs=[
            pl.BlockSpec(memory_space=pltpu.SMEM),
            pl.BlockSpec((_BN, d), lambda i: (i, 0)),
            pl.BlockSpec(memory_space=pl.ANY),
            pl.BlockSpec(memory_space=pltpu.VMEM),
        ],
        out_specs=[
            pl.BlockSpec((_BN, d), lambda i: (i, 0)),
            pl.BlockSpec(memory_space=pltpu.SMEM),
        ],
        out_shape=[
            jax.ShapeDtypeStruct((n, d), jnp.float32),
            jax.ShapeDtypeStruct((e,), jnp.float32),
        ],
        scratch_shapes=[
            pltpu.VMEM((d, d), jnp.float32),
            pltpu.VMEM((d, d), jnp.float32),
            pltpu.VMEM((d, d), jnp.bfloat16),
            pltpu.VMEM((1, d), jnp.float32),
            pltpu.SemaphoreType.DMA,
            pltpu.SemaphoreType.DMA,
        ],
    )(sp, x, Ws, bs)


def kernel(x, Ws, bs, scaling_params):
    e, d, _ = Ws.shape
    sp = scaling_params.astype(jnp.float32)
    sp16 = jnp.full((_LANES,), -jnp.inf, dtype=jnp.float32)
    sp16 = sp16.at[:e].set(sp)

    del sp16
    out, backbone_scores = _tc_mixture(x, Ws, bs, sp)
    return out, backbone_scores.astype(x.dtype)
